# Initial kernel scaffold; baseline (speedup 1.0000x reference)
import jax
import jax.numpy as jnp
from jax.experimental import pallas as pl


def kernel(inputs, lookup_table):
    raise NotImplementedError("write your pallas kernel here")


# speedup vs baseline: 1.3388x; 1.3388x over previous
import functools
import jax
import jax.numpy as jnp
from jax import lax
from jax.experimental import pallas as pl
from jax.experimental.pallas import tpu as pltpu
from jax.experimental.pallas import tpu_sc as plsc

H_UNITS_ = 32
SCALE_ = float(H_UNITS_ ** 0.5)

_NC = 2
_NS = 16
_NW = _NC * _NS
_LANES = 16

_SUB = 128
_NSUB_MACRO = 8
_MACRO = _SUB * _NSUB_MACRO


def _make_gather(n_rows_total: int, d: int):
    n_sub_total = n_rows_total // _SUB
    sub_per_w = n_sub_total // _NW
    n_macro = sub_per_w // _NSUB_MACRO

    mesh = plsc.VectorSubcoreMesh(core_axis_name="c", subcore_axis_name="s")

    @functools.partial(
        pl.kernel,
        mesh=mesh,
        compiler_params=pltpu.CompilerParams(use_tc_tiling_on_sc=False),
        out_type=jax.ShapeDtypeStruct((n_sub_total, _SUB, d), jnp.float32),
        scratch_types=[
            pltpu.VMEM((sub_per_w, _SUB), jnp.int32),
            pltpu.VMEM((_NSUB_MACRO, _SUB, d), jnp.float32),
            pltpu.SemaphoreType.DMA,
        ],
    )
    def k(table_hbm, idx_hbm, out_hbm, idx_v, rows_v, sem):
        wid = lax.axis_index("s") * _NC + lax.axis_index("c")
        sub_base = wid * sub_per_w
        pltpu.sync_copy(idx_hbm.at[pl.ds(sub_base, sub_per_w)], idx_v)

        def macro_body(m, carry):
            copies = []
            for j in range(_NSUB_MACRO):
                copies.append(
                    pltpu.async_copy(
                        table_hbm.at[idx_v.at[m * _NSUB_MACRO + j]],
                        rows_v.at[j],
                        sem,
                    )
                )
            for c in copies:
                c.wait()

            def scale_body(r, carry2):
                j = r // _SUB
                t = r % _SUB
                for h in range(d // _LANES):
                    sl = pl.ds(h * _LANES, _LANES)
                    rows_v[j, t, sl] = rows_v[j, t, sl] * SCALE_
                return carry2

            lax.fori_loop(0, _MACRO, scale_body, 0, unroll=2)

            pltpu.sync_copy(
                rows_v,
                out_hbm.at[pl.ds(sub_base + m * _NSUB_MACRO, _NSUB_MACRO)],
            )
            return carry

        lax.fori_loop(0, n_macro, macro_body, 0)

    return k


def kernel(inputs, lookup_table):
    b0, b1 = inputs.shape
    n = b0 * b1
    d = lookup_table.shape[1]
    idx = inputs.reshape(n // _SUB, _SUB).astype(jnp.int32)
    out = _make_gather(n, d)(lookup_table, idx)
    return out.reshape(b0, b1, d)
